# baseline (speedup 1.0000x reference)
import functools

import jax
import jax.numpy as jnp
from jax import lax
from jax.experimental import pallas as pl
from jax.experimental.pallas import tpu as pltpu
from jax.experimental.pallas import tpu_sc as plsc

MAX_DEGREE = 64
EMB_DIM = 128
N_NODES = 100000
N_EDGES = 3200000

NC = 2
NS = 16
NW = NC * NS

ZCHUNK = 2048
N_ZCHUNKS = 49
HIST_PAD = ZCHUNK * N_ZCHUNKS
N_CHUNKS = 782
TAIL_CHUNK = N_CHUNKS - 1
TAIL_NODES = N_NODES - TAIL_CHUNK * 128

EDGE_ROWS = N_EDGES // 128
ROWS_PER_BLK = 8
N_BLKS = EDGE_ROWS // ROWS_PER_BLK
BLK_ITERS = (N_BLKS + NW - 1) // NW

_mesh = plsc.VectorSubcoreMesh(core_axis_name="c", subcore_axis_name="s")


@functools.partial(
    pl.kernel,
    out_type=jax.ShapeDtypeStruct((NC * HIST_PAD,), jnp.int32),
    mesh=_mesh,
    scratch_types=[
        pltpu.VMEM_SHARED((HIST_PAD,), jnp.int32),
        pltpu.VMEM((ROWS_PER_BLK, 128), jnp.int32),
        pltpu.VMEM((128,), jnp.int32),
        pltpu.VMEM((ZCHUNK,), jnp.int32),
    ],
)
def _hist_kernel(src_hbm, hist_hbm, hist_s, idx_v, ones_v, zero_v):
    cid = lax.axis_index("c")
    sid = lax.axis_index("s")
    w = cid * NS + sid

    for i in range(8):
        ones_v[pl.ds(i * 16, 16)] = jnp.full((16,), 1, jnp.int32)
    for i in range(ZCHUNK // 16):
        zero_v[pl.ds(i * 16, 16)] = jnp.zeros((16,), jnp.int32)

    for j in range((N_ZCHUNKS + NS - 1) // NS):
        c = sid + j * NS

        @pl.when(c < N_ZCHUNKS)
        def _():
            pltpu.sync_copy(zero_v, hist_s.at[pl.ds(c * ZCHUNK, ZCHUNK)])

    plsc.subcore_barrier()

    def body(b, _):
        blk = w + b * NW

        @pl.when(blk < N_BLKS)
        def _():
            pltpu.sync_copy(src_hbm.at[pl.ds(blk * ROWS_PER_BLK, ROWS_PER_BLK), :],
                            idx_v)
            for j in range(ROWS_PER_BLK):
                pltpu.sync_copy(ones_v, hist_s.at[idx_v.at[j]], add=True)

        return _

    lax.fori_loop(0, BLK_ITERS, body, None)

    plsc.subcore_barrier()

    @pl.when(sid == 0)
    def _():
        pltpu.sync_copy(hist_s, hist_hbm.at[pl.ds(cid * HIST_PAD, HIST_PAD)])


@functools.partial(
    pl.kernel,
    out_type=jax.ShapeDtypeStruct((N_NODES, EMB_DIM), jnp.float32),
    mesh=_mesh,
    scratch_types=[
        pltpu.VMEM((128,), jnp.int32),
        pltpu.VMEM((128,), jnp.int32),
        pltpu.VMEM((128,), jnp.int32),
        pltpu.VMEM((128, EMB_DIM), jnp.float32),
        pltpu.SemaphoreType.DMA,
    ],
)
def _encode_kernel(hist_hbm, emb_hbm, out_hbm, h0_v, h1_v, idx_v, rows_v, sem):
    cid = lax.axis_index("c")
    sid = lax.axis_index("s")
    w = cid * NS + sid

    def body(b, _):
        c = w + b * NW

        @pl.when(c < N_CHUNKS)
        def _():
            base = c * 128
            pltpu.sync_copy(hist_hbm.at[pl.ds(base, 128)], h0_v)
            pltpu.sync_copy(hist_hbm.at[pl.ds(HIST_PAD + base, 128)], h1_v)
            for i in range(8):
                s = pl.ds(i * 16, 16)
                deg = h0_v[s] + h1_v[s]
                idx_v[s] = jnp.minimum(jnp.maximum(deg, 1), MAX_DEGREE) - 1
            pltpu.async_copy(emb_hbm.at[idx_v], rows_v, sem).wait()

            @pl.when(c < TAIL_CHUNK)
            def _():
                pltpu.sync_copy(rows_v, out_hbm.at[pl.ds(base, 128), :])

            @pl.when(c == TAIL_CHUNK)
            def _():
                pltpu.sync_copy(rows_v.at[pl.ds(0, TAIL_NODES), :],
                                out_hbm.at[pl.ds(base, TAIL_NODES), :])

        return _

    lax.fori_loop(0, (N_CHUNKS + NW - 1) // NW, body, None)


def kernel(edge_index, num_nodes, emb_weight):
    src = edge_index[0].reshape(EDGE_ROWS, 128)
    hist = _hist_kernel(src)
    return _encode_kernel(hist, emb_weight)


# speedup vs baseline: 1.0012x; 1.0012x over previous
import functools

import jax
import jax.numpy as jnp
from jax import lax
from jax.experimental import pallas as pl
from jax.experimental.pallas import tpu as pltpu
from jax.experimental.pallas import tpu_sc as plsc

MAX_DEGREE = 64
EMB_DIM = 128
N_NODES = 100000
N_EDGES = 3200000

NC = 2
NS = 16
NW = NC * NS

ZCHUNK = 2048
N_ZCHUNKS = 49
HIST_PAD = ZCHUNK * N_ZCHUNKS
N_CHUNKS = 782
TAIL_CHUNK = N_CHUNKS - 1
TAIL_NODES = N_NODES - TAIL_CHUNK * 128

EDGE_ROWS = N_EDGES // 128
ROWS_PER_BLK = 8
N_BLKS = EDGE_ROWS // ROWS_PER_BLK
BLK_ITERS = (N_BLKS + NW - 1) // NW

_mesh = plsc.VectorSubcoreMesh(core_axis_name="c", subcore_axis_name="s")


@functools.partial(
    pl.kernel,
    out_type=jax.ShapeDtypeStruct((NC * HIST_PAD,), jnp.int32),
    mesh=_mesh,
    scratch_types=[
        pltpu.VMEM_SHARED((HIST_PAD,), jnp.int32),
        pltpu.VMEM((ROWS_PER_BLK, 128), jnp.int32),
        pltpu.VMEM((128,), jnp.int32),
        pltpu.VMEM((ZCHUNK,), jnp.int32),
    ],
)
def _hist_kernel(src_hbm, hist_hbm, hist_s, idx_v, ones_v, zero_v):
    cid = lax.axis_index("c")
    sid = lax.axis_index("s")
    w = cid * NS + sid

    for i in range(8):
        ones_v[pl.ds(i * 16, 16)] = jnp.full((16,), 1, jnp.int32)
    for i in range(ZCHUNK // 16):
        zero_v[pl.ds(i * 16, 16)] = jnp.zeros((16,), jnp.int32)

    for j in range((N_ZCHUNKS + NS - 1) // NS):
        c = sid + j * NS

        @pl.when(c < N_ZCHUNKS)
        def _():
            pltpu.sync_copy(zero_v, hist_s.at[pl.ds(c * ZCHUNK, ZCHUNK)])

    plsc.subcore_barrier()

    def body(b, _):
        blk = w + b * NW

        @pl.when(blk < N_BLKS)
        def _():
            pltpu.sync_copy(src_hbm.at[pl.ds(blk * ROWS_PER_BLK, ROWS_PER_BLK), :],
                            idx_v)
            for j in range(ROWS_PER_BLK):
                pltpu.sync_copy(ones_v, hist_s.at[idx_v.at[j]], add=True)

        return _

    lax.fori_loop(0, BLK_ITERS, body, None)

    plsc.subcore_barrier()

    @pl.when(sid == 0)
    def _():
        pltpu.sync_copy(hist_s, hist_hbm.at[pl.ds(cid * HIST_PAD, HIST_PAD)])


MAX_WCHUNKS = 25
SLAB = MAX_WCHUNKS * 128
NBUF = 4
ENC_SLOTS = ((MAX_WCHUNKS + NBUF) // NBUF) * NBUF


@functools.partial(
    pl.kernel,
    out_type=jax.ShapeDtypeStruct((N_NODES, EMB_DIM), jnp.float32),
    mesh=_mesh,
    scratch_types=[
        pltpu.VMEM((SLAB,), jnp.int32),
        pltpu.VMEM((SLAB,), jnp.int32),
        pltpu.VMEM((SLAB,), jnp.int32),
        [pltpu.VMEM((128, EMB_DIM), jnp.float32) for _ in range(NBUF)],
        [pltpu.SemaphoreType.DMA for _ in range(NBUF)],
    ],
)
def _encode_kernel(hist_hbm, emb_hbm, out_hbm, h0_v, h1_v, idx_v, rows, gsem):
    cid = lax.axis_index("c")
    sid = lax.axis_index("s")
    w = cid * NS + sid
    start = w * 24 + jnp.minimum(w, 14)
    n = jnp.where(w < 14, 25, 24)

    pltpu.sync_copy(hist_hbm.at[pl.ds(start * 128, SLAB)], h0_v)
    pltpu.sync_copy(hist_hbm.at[pl.ds(HIST_PAD + start * 128, SLAB)], h1_v)

    def clip_body(i, _):
        s = pl.ds(i * 16, 16)
        deg = h0_v[s] + h1_v[s]
        idx_v[s] = jnp.minimum(jnp.maximum(deg, 1), MAX_DEGREE) - 1
        return _

    lax.fori_loop(0, SLAB // 16, clip_body, None)

    def _gather(l, k):
        pltpu.async_copy(emb_hbm.at[idx_v.at[pl.ds(l * 128, 128)]], rows[k],
                         gsem[k])

    def _write(l, k):
        c = start + l
        base = c * 128

        @pl.when(c < TAIL_CHUNK)
        def _():
            pltpu.sync_copy(rows[k], out_hbm.at[pl.ds(base, 128), :])

        @pl.when(c == TAIL_CHUNK)
        def _():
            pltpu.sync_copy(rows[k].at[pl.ds(0, TAIL_NODES), :],
                            out_hbm.at[pl.ds(base, TAIL_NODES), :])

    for k in range(NBUF):
        @pl.when(k < n)
        def _(k=k):
            _gather(k, k)

    def body(j, _):
        for k in range(NBUF):
            l = j * NBUF + k

            @pl.when(l < n)
            def _(k=k, l=l):
                pltpu.make_async_copy(
                    emb_hbm.at[idx_v.at[pl.ds(l * 128, 128)]], rows[k],
                    gsem[k]).wait()
                _write(l, k)

                @pl.when(l + NBUF < n)
                def _():
                    _gather(l + NBUF, k)

        return _

    lax.fori_loop(0, ENC_SLOTS // NBUF, body, None)


def kernel(edge_index, num_nodes, emb_weight):
    src = edge_index[0].reshape(EDGE_ROWS, 128)
    hist = _hist_kernel(src)
    return _encode_kernel(hist, emb_weight)


# speedup vs baseline: 1.1697x; 1.1683x over previous
import functools

import jax
import jax.numpy as jnp
from jax import lax
from jax.experimental import pallas as pl
from jax.experimental.pallas import tpu as pltpu
from jax.experimental.pallas import tpu_sc as plsc

MAX_DEGREE = 64
EMB_DIM = 128
N_NODES = 100000
N_EDGES = 3200000

NC = 2
NS = 16
NW = NC * NS

ZCHUNK = 2048
N_ZCHUNKS = 49
HIST_PAD = ZCHUNK * N_ZCHUNKS
N_CHUNKS = 782
TAIL_CHUNK = N_CHUNKS - 1
TAIL_NODES = N_NODES - TAIL_CHUNK * 128

EDGE_ROWS = N_EDGES // 128
RPB = 8
N_BLKS = EDGE_ROWS // RPB
BLK_ITERS = (N_BLKS + NW - 1) // NW

_mesh = plsc.VectorSubcoreMesh(core_axis_name="c", subcore_axis_name="s")

SLICE = HIST_PAD // NS

NB_H = 6
LA_H = 2


@functools.partial(
    pl.kernel,
    out_type=jax.ShapeDtypeStruct((NC * HIST_PAD,), jnp.int32),
    mesh=_mesh,
    scratch_types=[
        pltpu.VMEM_SHARED((HIST_PAD,), jnp.int32),
        [pltpu.VMEM((RPB, 128), jnp.int32) for _ in range(NB_H)],
        [pltpu.SemaphoreType.DMA for _ in range(NB_H)],
        [pltpu.SemaphoreType.DMA for _ in range(NB_H)],
        pltpu.VMEM((128,), jnp.int32),
        pltpu.VMEM((ZCHUNK,), jnp.int32),
    ],
)
def _hist_kernel(src_hbm, hist_hbm, hist_s, ebuf, esem, ssem, ones_v, zero_v):
    cid = lax.axis_index("c")
    sid = lax.axis_index("s")
    w = cid * NS + sid

    for i in range(8):
        ones_v[pl.ds(i * 16, 16)] = jnp.full((16,), 1, jnp.int32)
    for i in range(ZCHUNK // 16):
        zero_v[pl.ds(i * 16, 16)] = jnp.zeros((16,), jnp.int32)

    for j in range((N_ZCHUNKS + NS - 1) // NS):
        c = sid + j * NS

        @pl.when(c < N_ZCHUNKS)
        def _():
            pltpu.sync_copy(zero_v, hist_s.at[pl.ds(c * ZCHUNK, ZCHUNK)])

    plsc.subcore_barrier()

    def _valid(b):
        return w + b * NW < N_BLKS

    def _load(b, k):
        pltpu.async_copy(
            src_hbm.at[pl.ds((w + b * NW) * RPB, RPB), :], ebuf[k], esem[k])

    def _wait_load(b, k):
        pltpu.make_async_copy(
            src_hbm.at[pl.ds((w + b * NW) * RPB, RPB), :], ebuf[k],
            esem[k]).wait()

    def _scatters(k):
        for r in range(RPB):
            pltpu.async_copy(ones_v, hist_s.at[ebuf[k].at[r]], ssem[k],
                             add=True)

    def _drain(k):
        for r in range(RPB):
            pltpu.make_async_copy(ones_v, hist_s.at[ebuf[k].at[r]],
                                  ssem[k]).wait()

    for s in range(LA_H):
        @pl.when(_valid(s))
        def _(s=s):
            _load(s, s % NB_H)

    def body(j, _):
        for k0 in range(NB_H):
            s = j * NB_H + k0 + LA_H
            k = (k0 + LA_H) % NB_H
            kp = k0

            @pl.when(s < BLK_ITERS)
            def _(s=s, k=k):
                bd = s - NB_H

                @pl.when((bd >= 0) & _valid(bd))
                def _():
                    _drain(k)

                @pl.when(_valid(s))
                def _():
                    _load(s, k)

            bp = s - LA_H

            @pl.when((bp < BLK_ITERS) & _valid(bp))
            def _(bp=bp, kp=kp):
                _wait_load(bp, kp)
                _scatters(kp)

        return _

    lax.fori_loop(0, (BLK_ITERS + NB_H - 1) // NB_H + 1, body, None)

    for bd in range(BLK_ITERS - NB_H, BLK_ITERS):
        @pl.when(_valid(bd))
        def _(bd=bd):
            _drain(bd % NB_H)

    plsc.subcore_barrier()

    pltpu.sync_copy(hist_s.at[pl.ds(sid * SLICE, SLICE)],
                    hist_hbm.at[pl.ds(cid * HIST_PAD + sid * SLICE, SLICE)])


MAX_WCHUNKS = 25
SLAB = MAX_WCHUNKS * 128
NBUF = 4
ENC_SLOTS = ((MAX_WCHUNKS + NBUF) // NBUF) * NBUF


@functools.partial(
    pl.kernel,
    out_type=jax.ShapeDtypeStruct((N_NODES, EMB_DIM), jnp.float32),
    mesh=_mesh,
    scratch_types=[
        pltpu.VMEM((SLAB,), jnp.int32),
        pltpu.VMEM((SLAB,), jnp.int32),
        pltpu.VMEM((SLAB,), jnp.int32),
        [pltpu.VMEM((128, EMB_DIM), jnp.float32) for _ in range(NBUF)],
        [pltpu.SemaphoreType.DMA for _ in range(NBUF)],
    ],
)
def _encode_kernel(hist_hbm, emb_hbm, out_hbm, h0_v, h1_v, idx_v, rows, gsem):
    cid = lax.axis_index("c")
    sid = lax.axis_index("s")
    w = cid * NS + sid
    start = w * 24 + jnp.minimum(w, 14)
    n = jnp.where(w < 14, 25, 24)

    pltpu.sync_copy(hist_hbm.at[pl.ds(start * 128, SLAB)], h0_v)
    pltpu.sync_copy(hist_hbm.at[pl.ds(HIST_PAD + start * 128, SLAB)], h1_v)

    def clip_body(i, _):
        s = pl.ds(i * 16, 16)
        deg = h0_v[s] + h1_v[s]
        idx_v[s] = jnp.minimum(jnp.maximum(deg, 1), MAX_DEGREE) - 1
        return _

    lax.fori_loop(0, SLAB // 16, clip_body, None)

    def _gather(l, k):
        pltpu.async_copy(emb_hbm.at[idx_v.at[pl.ds(l * 128, 128)]], rows[k],
                         gsem[k])

    def _write(l, k):
        c = start + l
        base = c * 128

        @pl.when(c < TAIL_CHUNK)
        def _():
            pltpu.sync_copy(rows[k], out_hbm.at[pl.ds(base, 128), :])

        @pl.when(c == TAIL_CHUNK)
        def _():
            pltpu.sync_copy(rows[k].at[pl.ds(0, TAIL_NODES), :],
                            out_hbm.at[pl.ds(base, TAIL_NODES), :])

    for k in range(NBUF):
        @pl.when(k < n)
        def _(k=k):
            _gather(k, k)

    def body(j, _):
        for k in range(NBUF):
            l = j * NBUF + k

            @pl.when(l < n)
            def _(k=k, l=l):
                pltpu.make_async_copy(
                    emb_hbm.at[idx_v.at[pl.ds(l * 128, 128)]], rows[k],
                    gsem[k]).wait()
                _write(l, k)

                @pl.when(l + NBUF < n)
                def _():
                    _gather(l + NBUF, k)

        return _

    lax.fori_loop(0, ENC_SLOTS // NBUF, body, None)


def kernel(edge_index, num_nodes, emb_weight):
    src = edge_index[0].reshape(EDGE_ROWS, 128)
    hist = _hist_kernel(src)
    return _encode_kernel(hist, emb_weight)


# speedup vs baseline: 8.2435x; 7.0475x over previous
import functools

import jax
import jax.numpy as jnp
from jax import lax
from jax.experimental import pallas as pl
from jax.experimental.pallas import tpu as pltpu
from jax.experimental.pallas import tpu_sc as plsc

MAX_DEGREE = 64
EMB_DIM = 128
N_NODES = 100000
N_EDGES = 3200000

NC = 2
NS = 16
NW = NC * NS

ZCHUNK = 2048
N_ZCHUNKS = 49
HIST_PAD = ZCHUNK * N_ZCHUNKS
N_CHUNKS = 782
TAIL_CHUNK = N_CHUNKS - 1
TAIL_NODES = N_NODES - TAIL_CHUNK * 128

EDGE_ROWS = N_EDGES // 128
RPB = 8
N_BLKS = EDGE_ROWS // RPB
BLK_ITERS = (N_BLKS + NW - 1) // NW

_mesh = plsc.VectorSubcoreMesh(core_axis_name="c", subcore_axis_name="s")

SLICE = HIST_PAD // NS

NB_H = 6
LA_H = 2


@functools.partial(
    pl.kernel,
    out_type=jax.ShapeDtypeStruct((NC * HIST_PAD,), jnp.int32),
    mesh=_mesh,
    scratch_types=[
        pltpu.VMEM_SHARED((HIST_PAD,), jnp.int32),
        [pltpu.VMEM((RPB, 128), jnp.int32) for _ in range(NB_H)],
        [pltpu.SemaphoreType.DMA for _ in range(NB_H)],
        [pltpu.SemaphoreType.DMA for _ in range(NB_H)],
        pltpu.VMEM((128,), jnp.int32),
        pltpu.VMEM((ZCHUNK,), jnp.int32),
    ],
)
def _hist_kernel(src_hbm, hist_hbm, hist_s, ebuf, esem, ssem, ones_v, zero_v):
    cid = lax.axis_index("c")
    sid = lax.axis_index("s")
    w = cid * NS + sid

    for i in range(8):
        ones_v[pl.ds(i * 16, 16)] = jnp.full((16,), 1, jnp.int32)
    for i in range(ZCHUNK // 16):
        zero_v[pl.ds(i * 16, 16)] = jnp.zeros((16,), jnp.int32)

    for j in range((N_ZCHUNKS + NS - 1) // NS):
        c = sid + j * NS

        @pl.when(c < N_ZCHUNKS)
        def _():
            pltpu.sync_copy(zero_v, hist_s.at[pl.ds(c * ZCHUNK, ZCHUNK)])

    plsc.subcore_barrier()

    def _valid(b):
        return w + b * NW < N_BLKS

    def _load(b, k):
        pltpu.async_copy(
            src_hbm.at[pl.ds((w + b * NW) * RPB, RPB), :], ebuf[k], esem[k])

    def _wait_load(b, k):
        pltpu.make_async_copy(
            src_hbm.at[pl.ds((w + b * NW) * RPB, RPB), :], ebuf[k],
            esem[k]).wait()

    def _scatters(k):
        for r in range(RPB):
            pltpu.async_copy(ones_v, hist_s.at[ebuf[k].at[r]], ssem[k],
                             add=True)

    def _drain(k):
        for r in range(RPB):
            pltpu.make_async_copy(ones_v, hist_s.at[ebuf[k].at[r]],
                                  ssem[k]).wait()

    for s in range(LA_H):
        @pl.when(_valid(s))
        def _(s=s):
            _load(s, s % NB_H)

    def body(j, _):
        for k0 in range(NB_H):
            s = j * NB_H + k0 + LA_H
            k = (k0 + LA_H) % NB_H
            kp = k0

            @pl.when(s < BLK_ITERS)
            def _(s=s, k=k):
                bd = s - NB_H

                @pl.when((bd >= 0) & _valid(bd))
                def _():
                    _drain(k)

                @pl.when(_valid(s))
                def _():
                    _load(s, k)

            bp = s - LA_H

            @pl.when((bp < BLK_ITERS) & _valid(bp))
            def _(bp=bp, kp=kp):
                _wait_load(bp, kp)
                _scatters(kp)

        return _

    lax.fori_loop(0, (BLK_ITERS + NB_H - 1) // NB_H + 1, body, None)

    for bd in range(BLK_ITERS - NB_H, BLK_ITERS):
        @pl.when(_valid(bd))
        def _(bd=bd):
            _drain(bd % NB_H)

    plsc.subcore_barrier()

    pltpu.sync_copy(hist_s.at[pl.ds(sid * SLICE, SLICE)],
                    hist_hbm.at[pl.ds(cid * HIST_PAD + sid * SLICE, SLICE)])


MAX_WCHUNKS = 25
SLAB = MAX_WCHUNKS * 128
NBUF = 4
ENC_SLOTS = ((MAX_WCHUNKS + NBUF) // NBUF) * NBUF


@functools.partial(
    pl.kernel,
    out_type=jax.ShapeDtypeStruct((N_NODES, EMB_DIM), jnp.float32),
    mesh=_mesh,
    scratch_types=[
        pltpu.VMEM((SLAB,), jnp.int32),
        pltpu.VMEM((SLAB,), jnp.int32),
        pltpu.VMEM((SLAB,), jnp.int32),
        pltpu.VMEM_SHARED((MAX_DEGREE, EMB_DIM), jnp.float32),
        [pltpu.VMEM((128, EMB_DIM), jnp.float32) for _ in range(NBUF)],
        [pltpu.SemaphoreType.DMA for _ in range(NBUF)],
    ],
)
def _encode_kernel(hist_hbm, emb_hbm, out_hbm, h0_v, h1_v, idx_v, table_s,
                   rows, gsem):
    cid = lax.axis_index("c")
    sid = lax.axis_index("s")
    w = cid * NS + sid
    start = w * 24 + jnp.minimum(w, 14)
    n = jnp.where(w < 14, 25, 24)

    @pl.when(sid == 0)
    def _():
        pltpu.sync_copy(emb_hbm, table_s)

    pltpu.sync_copy(hist_hbm.at[pl.ds(start * 128, SLAB)], h0_v)
    pltpu.sync_copy(hist_hbm.at[pl.ds(HIST_PAD + start * 128, SLAB)], h1_v)

    def clip_body(i, _):
        s = pl.ds(i * 16, 16)
        deg = h0_v[s] + h1_v[s]
        idx_v[s] = jnp.minimum(jnp.maximum(deg, 1), MAX_DEGREE) - 1
        return _

    lax.fori_loop(0, SLAB // 16, clip_body, None)

    def _gather(l, k):
        pltpu.async_copy(table_s.at[idx_v.at[pl.ds(l * 128, 128)]], rows[k],
                         gsem[k])

    def _write(l, k):
        c = start + l
        base = c * 128

        @pl.when(c < TAIL_CHUNK)
        def _():
            pltpu.sync_copy(rows[k], out_hbm.at[pl.ds(base, 128), :])

        @pl.when(c == TAIL_CHUNK)
        def _():
            pltpu.sync_copy(rows[k].at[pl.ds(0, TAIL_NODES), :],
                            out_hbm.at[pl.ds(base, TAIL_NODES), :])

    plsc.subcore_barrier()

    for k in range(NBUF):
        @pl.when(k < n)
        def _(k=k):
            _gather(k, k)

    def body(j, _):
        for k in range(NBUF):
            l = j * NBUF + k

            @pl.when(l < n)
            def _(k=k, l=l):
                pltpu.make_async_copy(
                    table_s.at[idx_v.at[pl.ds(l * 128, 128)]], rows[k],
                    gsem[k]).wait()
                _write(l, k)

                @pl.when(l + NBUF < n)
                def _():
                    _gather(l + NBUF, k)

        return _

    lax.fori_loop(0, ENC_SLOTS // NBUF, body, None)


def kernel(edge_index, num_nodes, emb_weight):
    src = edge_index[0].reshape(EDGE_ROWS, 128)
    hist = _hist_kernel(src)
    return _encode_kernel(hist, emb_weight)


# speedup vs baseline: 8.2946x; 1.0062x over previous
import functools

import jax
import jax.numpy as jnp
from jax import lax
from jax.experimental import pallas as pl
from jax.experimental.pallas import tpu as pltpu
from jax.experimental.pallas import tpu_sc as plsc

MAX_DEGREE = 64
EMB_DIM = 128
N_NODES = 100000
N_EDGES = 3200000

NC = 2
NS = 16
NW = NC * NS

ZCHUNK = 2048
N_ZCHUNKS = 49
HIST_PAD = ZCHUNK * N_ZCHUNKS
N_CHUNKS = 782
TAIL_CHUNK = N_CHUNKS - 1
TAIL_NODES = N_NODES - TAIL_CHUNK * 128

EDGE_ROWS = N_EDGES // 128
RPB = 8
N_BLKS = EDGE_ROWS // RPB
BLK_ITERS = (N_BLKS + NW - 1) // NW

_mesh = plsc.VectorSubcoreMesh(core_axis_name="c", subcore_axis_name="s")

SLICE = HIST_PAD // NS

NB_H = 10
LA_H = 3


@functools.partial(
    pl.kernel,
    out_type=jax.ShapeDtypeStruct((NC * HIST_PAD,), jnp.int32),
    mesh=_mesh,
    scratch_types=[
        pltpu.VMEM_SHARED((HIST_PAD,), jnp.int32),
        [pltpu.VMEM((RPB, 128), jnp.int32) for _ in range(NB_H)],
        [pltpu.SemaphoreType.DMA for _ in range(NB_H)],
        [pltpu.SemaphoreType.DMA for _ in range(NB_H)],
        pltpu.VMEM((128,), jnp.int32),
        pltpu.VMEM((ZCHUNK,), jnp.int32),
    ],
)
def _hist_kernel(src_hbm, hist_hbm, hist_s, ebuf, esem, ssem, ones_v, zero_v):
    cid = lax.axis_index("c")
    sid = lax.axis_index("s")
    w = cid * NS + sid

    def _valid(b):
        return w + b * NW < N_BLKS

    def _load(b, k):
        pltpu.async_copy(
            src_hbm.at[pl.ds((w + b * NW) * RPB, RPB), :], ebuf[k], esem[k])

    for s in range(LA_H):
        @pl.when(_valid(s))
        def _(s=s):
            _load(s, s % NB_H)

    for i in range(8):
        ones_v[pl.ds(i * 16, 16)] = jnp.full((16,), 1, jnp.int32)
    for i in range(ZCHUNK // 16):
        zero_v[pl.ds(i * 16, 16)] = jnp.zeros((16,), jnp.int32)

    for j in range((N_ZCHUNKS + NS - 1) // NS):
        c = sid + j * NS

        @pl.when(c < N_ZCHUNKS)
        def _():
            pltpu.sync_copy(zero_v, hist_s.at[pl.ds(c * ZCHUNK, ZCHUNK)])

    plsc.subcore_barrier()

    def _wait_load(b, k):
        pltpu.make_async_copy(
            src_hbm.at[pl.ds((w + b * NW) * RPB, RPB), :], ebuf[k],
            esem[k]).wait()

    def _scatters(k):
        for r in range(RPB):
            pltpu.async_copy(ones_v, hist_s.at[ebuf[k].at[r]], ssem[k],
                             add=True)

    def _drain(k):
        for r in range(RPB):
            pltpu.make_async_copy(ones_v, hist_s.at[ebuf[k].at[r]],
                                  ssem[k]).wait()

    def body(j, _):
        for k0 in range(NB_H):
            s = j * NB_H + k0 + LA_H
            k = (k0 + LA_H) % NB_H
            kp = k0

            @pl.when(s < BLK_ITERS)
            def _(s=s, k=k):
                bd = s - NB_H

                @pl.when((bd >= 0) & _valid(bd))
                def _():
                    _drain(k)

                @pl.when(_valid(s))
                def _():
                    _load(s, k)

            bp = s - LA_H

            @pl.when((bp < BLK_ITERS) & _valid(bp))
            def _(bp=bp, kp=kp):
                _wait_load(bp, kp)
                _scatters(kp)

        return _

    lax.fori_loop(0, (BLK_ITERS + NB_H - 1) // NB_H + 1, body, None)

    for bd in range(BLK_ITERS - NB_H, BLK_ITERS):
        @pl.when(_valid(bd))
        def _(bd=bd):
            _drain(bd % NB_H)

    plsc.subcore_barrier()

    pltpu.sync_copy(hist_s.at[pl.ds(sid * SLICE, SLICE)],
                    hist_hbm.at[pl.ds(cid * HIST_PAD + sid * SLICE, SLICE)])


MAX_WCHUNKS = 25
SLAB = MAX_WCHUNKS * 128
NBUF = 4
ENC_SLOTS = ((MAX_WCHUNKS + NBUF) // NBUF) * NBUF


@functools.partial(
    pl.kernel,
    out_type=jax.ShapeDtypeStruct((N_NODES, EMB_DIM), jnp.float32),
    mesh=_mesh,
    scratch_types=[
        pltpu.VMEM((SLAB,), jnp.int32),
        pltpu.VMEM((SLAB,), jnp.int32),
        pltpu.VMEM((SLAB,), jnp.int32),
        pltpu.VMEM_SHARED((MAX_DEGREE, EMB_DIM), jnp.float32),
        [pltpu.VMEM((128, EMB_DIM), jnp.float32) for _ in range(NBUF)],
        [pltpu.SemaphoreType.DMA for _ in range(NBUF)],
    ],
)
def _encode_kernel(hist_hbm, emb_hbm, out_hbm, h0_v, h1_v, idx_v, table_s,
                   rows, gsem):
    cid = lax.axis_index("c")
    sid = lax.axis_index("s")
    w = cid * NS + sid
    start = w * 24 + jnp.minimum(w, 14)
    n = jnp.where(w < 14, 25, 24)

    @pl.when(sid == 0)
    def _():
        pltpu.sync_copy(emb_hbm, table_s)

    pltpu.sync_copy(hist_hbm.at[pl.ds(start * 128, SLAB)], h0_v)
    pltpu.sync_copy(hist_hbm.at[pl.ds(HIST_PAD + start * 128, SLAB)], h1_v)

    def clip_body(i, _):
        s = pl.ds(i * 16, 16)
        deg = h0_v[s] + h1_v[s]
        idx_v[s] = jnp.minimum(jnp.maximum(deg, 1), MAX_DEGREE) - 1
        return _

    lax.fori_loop(0, SLAB // 16, clip_body, None)

    def _gather(l, k):
        pltpu.async_copy(table_s.at[idx_v.at[pl.ds(l * 128, 128)]], rows[k],
                         gsem[k])

    def _write(l, k):
        c = start + l
        base = c * 128

        @pl.when(c < TAIL_CHUNK)
        def _():
            pltpu.sync_copy(rows[k], out_hbm.at[pl.ds(base, 128), :])

        @pl.when(c == TAIL_CHUNK)
        def _():
            pltpu.sync_copy(rows[k].at[pl.ds(0, TAIL_NODES), :],
                            out_hbm.at[pl.ds(base, TAIL_NODES), :])

    plsc.subcore_barrier()

    for k in range(NBUF):
        @pl.when(k < n)
        def _(k=k):
            _gather(k, k)

    def body(j, _):
        for k in range(NBUF):
            l = j * NBUF + k

            @pl.when(l < n)
            def _(k=k, l=l):
                pltpu.make_async_copy(
                    table_s.at[idx_v.at[pl.ds(l * 128, 128)]], rows[k],
                    gsem[k]).wait()
                _write(l, k)

                @pl.when(l + NBUF < n)
                def _():
                    _gather(l + NBUF, k)

        return _

    lax.fori_loop(0, ENC_SLOTS // NBUF, body, None)


def kernel(edge_index, num_nodes, emb_weight):
    src = edge_index[0].reshape(EDGE_ROWS, 128)
    hist = _hist_kernel(src)
    return _encode_kernel(hist, emb_weight)


# speedup vs baseline: 8.3210x; 1.0032x over previous
import functools

import jax
import jax.numpy as jnp
from jax import lax
from jax.experimental import pallas as pl
from jax.experimental.pallas import tpu as pltpu
from jax.experimental.pallas import tpu_sc as plsc

MAX_DEGREE = 64
EMB_DIM = 128
N_NODES = 100000
N_EDGES = 3200000

NC = 2
NS = 16
NW = NC * NS

ZCHUNK = 2048
N_ZCHUNKS = 49
HIST_PAD = ZCHUNK * N_ZCHUNKS
N_CHUNKS = 782
TAIL_CHUNK = N_CHUNKS - 1
TAIL_NODES = N_NODES - TAIL_CHUNK * 128

BLK_E = 1024
N_BLKS = N_EDGES // BLK_E
BLK_ITERS = (N_BLKS + NW - 1) // NW

_mesh = plsc.VectorSubcoreMesh(core_axis_name="c", subcore_axis_name="s")

SLICE = HIST_PAD // NS

NB_H = 10
LA_H = 3


@functools.partial(
    pl.kernel,
    out_type=jax.ShapeDtypeStruct((NC * HIST_PAD,), jnp.int32),
    mesh=_mesh,
    scratch_types=[
        pltpu.VMEM_SHARED((HIST_PAD,), jnp.int32),
        [pltpu.VMEM((BLK_E,), jnp.int32) for _ in range(NB_H)],
        [pltpu.SemaphoreType.DMA for _ in range(NB_H)],
        [pltpu.SemaphoreType.DMA for _ in range(NB_H)],
        pltpu.VMEM((BLK_E,), jnp.int32),
        pltpu.VMEM((ZCHUNK,), jnp.int32),
    ],
)
def _hist_kernel(src_hbm, hist_hbm, hist_s, ebuf, esem, ssem, ones_v, zero_v):
    cid = lax.axis_index("c")
    sid = lax.axis_index("s")
    w = cid * NS + sid

    def _valid(b):
        return w + b * NW < N_BLKS

    def _load(b, k):
        pltpu.async_copy(
            src_hbm.at[pl.ds((w + b * NW) * BLK_E, BLK_E)], ebuf[k], esem[k])

    for s in range(LA_H):
        @pl.when(_valid(s))
        def _(s=s):
            _load(s, s % NB_H)

    for i in range(BLK_E // 16):
        ones_v[pl.ds(i * 16, 16)] = jnp.full((16,), 1, jnp.int32)
    for i in range(ZCHUNK // 16):
        zero_v[pl.ds(i * 16, 16)] = jnp.zeros((16,), jnp.int32)

    for j in range((N_ZCHUNKS + NS - 1) // NS):
        c = sid + j * NS

        @pl.when(c < N_ZCHUNKS)
        def _():
            pltpu.sync_copy(zero_v, hist_s.at[pl.ds(c * ZCHUNK, ZCHUNK)])

    plsc.subcore_barrier()

    def _wait_load(b, k):
        pltpu.make_async_copy(
            src_hbm.at[pl.ds((w + b * NW) * BLK_E, BLK_E)], ebuf[k],
            esem[k]).wait()

    def _scatters(k):
        pltpu.async_copy(ones_v, hist_s.at[ebuf[k]], ssem[k], add=True)

    def _drain(k):
        pltpu.make_async_copy(ones_v, hist_s.at[ebuf[k]], ssem[k]).wait()

    def body(j, _):
        for k0 in range(NB_H):
            s = j * NB_H + k0 + LA_H
            k = (k0 + LA_H) % NB_H
            kp = k0

            @pl.when(s < BLK_ITERS)
            def _(s=s, k=k):
                bd = s - NB_H

                @pl.when((bd >= 0) & _valid(bd))
                def _():
                    _drain(k)

                @pl.when(_valid(s))
                def _():
                    _load(s, k)

            bp = s - LA_H

            @pl.when((bp < BLK_ITERS) & _valid(bp))
            def _(bp=bp, kp=kp):
                _wait_load(bp, kp)
                _scatters(kp)

        return _

    lax.fori_loop(0, (BLK_ITERS + NB_H - 1) // NB_H + 1, body, None)

    for bd in range(BLK_ITERS - NB_H, BLK_ITERS):
        @pl.when(_valid(bd))
        def _(bd=bd):
            _drain(bd % NB_H)

    plsc.subcore_barrier()

    pltpu.sync_copy(hist_s.at[pl.ds(sid * SLICE, SLICE)],
                    hist_hbm.at[pl.ds(cid * HIST_PAD + sid * SLICE, SLICE)])


MAX_WCHUNKS = 25
SLAB = MAX_WCHUNKS * 128
NBUF = 4
ENC_SLOTS = ((MAX_WCHUNKS + NBUF) // NBUF) * NBUF


@functools.partial(
    pl.kernel,
    out_type=jax.ShapeDtypeStruct((N_NODES, EMB_DIM), jnp.float32),
    mesh=_mesh,
    scratch_types=[
        pltpu.VMEM((SLAB,), jnp.int32),
        pltpu.VMEM((SLAB,), jnp.int32),
        pltpu.VMEM((SLAB,), jnp.int32),
        pltpu.VMEM_SHARED((MAX_DEGREE, EMB_DIM), jnp.float32),
        [pltpu.VMEM((128, EMB_DIM), jnp.float32) for _ in range(NBUF)],
        [pltpu.SemaphoreType.DMA for _ in range(NBUF)],
    ],
)
def _encode_kernel(hist_hbm, emb_hbm, out_hbm, h0_v, h1_v, idx_v, table_s,
                   rows, gsem):
    cid = lax.axis_index("c")
    sid = lax.axis_index("s")
    w = cid * NS + sid
    start = w * 24 + jnp.minimum(w, 14)
    n = jnp.where(w < 14, 25, 24)

    @pl.when(sid == 0)
    def _():
        pltpu.sync_copy(emb_hbm, table_s)

    pltpu.sync_copy(hist_hbm.at[pl.ds(start * 128, SLAB)], h0_v)
    pltpu.sync_copy(hist_hbm.at[pl.ds(HIST_PAD + start * 128, SLAB)], h1_v)

    def clip_body(i, _):
        s = pl.ds(i * 16, 16)
        deg = h0_v[s] + h1_v[s]
        idx_v[s] = jnp.minimum(jnp.maximum(deg, 1), MAX_DEGREE) - 1
        return _

    lax.fori_loop(0, SLAB // 16, clip_body, None)

    def _gather(l, k):
        pltpu.async_copy(table_s.at[idx_v.at[pl.ds(l * 128, 128)]], rows[k],
                         gsem[k])

    def _write(l, k):
        c = start + l
        base = c * 128

        @pl.when(c < TAIL_CHUNK)
        def _():
            pltpu.sync_copy(rows[k], out_hbm.at[pl.ds(base, 128), :])

        @pl.when(c == TAIL_CHUNK)
        def _():
            pltpu.sync_copy(rows[k].at[pl.ds(0, TAIL_NODES), :],
                            out_hbm.at[pl.ds(base, TAIL_NODES), :])

    plsc.subcore_barrier()

    for k in range(NBUF):
        @pl.when(k < n)
        def _(k=k):
            _gather(k, k)

    def body(j, _):
        for k in range(NBUF):
            l = j * NBUF + k

            @pl.when(l < n)
            def _(k=k, l=l):
                pltpu.make_async_copy(
                    table_s.at[idx_v.at[pl.ds(l * 128, 128)]], rows[k],
                    gsem[k]).wait()
                _write(l, k)

                @pl.when(l + NBUF < n)
                def _():
                    _gather(l + NBUF, k)

        return _

    lax.fori_loop(0, ENC_SLOTS // NBUF, body, None)


def kernel(edge_index, num_nodes, emb_weight):
    hist = _hist_kernel(edge_index[0])
    return _encode_kernel(hist, emb_weight)
